# Initial kernel scaffold; baseline (speedup 1.0000x reference)
import jax
import jax.numpy as jnp
from jax.experimental import pallas as pl


def kernel(features, labels):
    raise NotImplementedError("write your pallas kernel here")


# speedup vs baseline: 15.4860x; 15.4860x over previous
import functools

import jax
import jax.numpy as jnp
from jax.experimental import pallas as pl

_TEMPERATURE = 0.07
_BASE_TEMPERATURE = 0.07


def _supcon_block_kernel(c_blk_ref, c_full_ref, lab_blk_ref, lab_full_ref,
                         out_ref, *, bm, n, inv_temp, scale):
    i = pl.program_id(0)
    c_blk = c_blk_ref[...]
    c_full = c_full_ref[...]
    s = jax.lax.dot_general(
        c_blk, c_full, (((1,), (1,)), ((), ())),
        preferred_element_type=jnp.float32) * inv_temp

    row_g = jax.lax.broadcasted_iota(jnp.int32, (bm, n), 0) + i * bm
    col = jax.lax.broadcasted_iota(jnp.int32, (bm, n), 1)
    not_diag = row_g != col

    m = jnp.max(s, axis=1, keepdims=True)
    p = jnp.where(not_diag, jnp.exp(s - m), 0.0)
    denom = jnp.sum(p, axis=1, keepdims=True)

    row_lab = lab_blk_ref[...].reshape(bm, 1)
    all_lab = lab_full_ref[...]
    mask = jnp.where((row_lab == all_lab) & not_diag, 1.0, 0.0)

    sum_mask = jnp.sum(mask, axis=1)
    sum_ms = jnp.sum(mask * s, axis=1)
    per_row = sum_ms / sum_mask - (m[:, 0] + jnp.log(denom[:, 0]))
    partial = jnp.reshape(-scale * jnp.sum(per_row), (1, 1))

    @pl.when(i == 0)
    def _init():
        out_ref[...] = partial

    @pl.when(i != 0)
    def _acc():
        out_ref[...] += partial


def kernel(features, labels):
    bsz, n_views, dim = features.shape
    n = bsz * n_views
    c = jnp.reshape(jnp.transpose(features, (1, 0, 2)), (n, dim))
    lab = jnp.reshape(jnp.tile(labels, n_views), (1, n))

    bm = 256
    grid = n // bm
    scale = (_TEMPERATURE / _BASE_TEMPERATURE) / n

    body = functools.partial(
        _supcon_block_kernel, bm=bm, n=n,
        inv_temp=1.0 / _TEMPERATURE, scale=scale)

    out = pl.pallas_call(
        body,
        grid=(grid,),
        in_specs=[
            pl.BlockSpec((bm, dim), lambda i: (i, 0)),
            pl.BlockSpec((n, dim), lambda i: (0, 0)),
            pl.BlockSpec((1, bm), lambda i: (0, i)),
            pl.BlockSpec((1, n), lambda i: (0, 0)),
        ],
        out_specs=pl.BlockSpec((1, 1), lambda i: (0, 0)),
        out_shape=jax.ShapeDtypeStruct((1, 1), jnp.float32),
    )(c, c, lab, lab)
    return out[0, 0]


# speedup vs baseline: 18.3803x; 1.1869x over previous
import functools

import jax
import jax.numpy as jnp
from jax.experimental import pallas as pl

_TEMPERATURE = 0.07
_BASE_TEMPERATURE = 0.07


def _supcon_block_kernel(c_blk_ref, c_full_ref, lab_blk_ref, lab_full_ref,
                         out_ref, *, bm, n, inv_temp, scale):
    i = pl.program_id(0)
    c_blk = c_blk_ref[...]
    c_full = c_full_ref[...]
    dot = jax.lax.dot_general(
        c_blk, c_full, (((1,), (1,)), ((), ())),
        preferred_element_type=jnp.float32)
    t = (dot - 1.0) * inv_temp

    sq = jnp.sum(c_blk * c_blk, axis=1)
    t_self = (sq - 1.0) * inv_temp

    p = jnp.exp(t)
    denom = jnp.sum(p, axis=1) - jnp.exp(t_self)

    row_lab = lab_blk_ref[...].reshape(bm, 1)
    all_lab = lab_full_ref[...]
    maskeq = row_lab == all_lab

    sum_mask = jnp.sum(jnp.where(maskeq, 1.0, 0.0), axis=1) - 1.0
    sum_mt = jnp.sum(jnp.where(maskeq, t, 0.0), axis=1) - t_self

    per_row = sum_mt / sum_mask - jnp.log(denom)
    partial = jnp.reshape(-scale * jnp.sum(per_row), (1, 1))

    @pl.when(i == 0)
    def _init():
        out_ref[...] = partial

    @pl.when(i != 0)
    def _acc():
        out_ref[...] += partial


def kernel(features, labels):
    bsz, n_views, dim = features.shape
    n = bsz * n_views
    c = jnp.reshape(jnp.transpose(features, (1, 0, 2)), (n, dim))
    lab = jnp.reshape(jnp.tile(labels, n_views), (1, n))

    bm = 256
    grid = n // bm
    scale = (_TEMPERATURE / _BASE_TEMPERATURE) / n

    body = functools.partial(
        _supcon_block_kernel, bm=bm, n=n,
        inv_temp=1.0 / _TEMPERATURE, scale=scale)

    out = pl.pallas_call(
        body,
        grid=(grid,),
        in_specs=[
            pl.BlockSpec((bm, dim), lambda i: (i, 0)),
            pl.BlockSpec((n, dim), lambda i: (0, 0)),
            pl.BlockSpec((1, bm), lambda i: (0, i)),
            pl.BlockSpec((1, n), lambda i: (0, 0)),
        ],
        out_specs=pl.BlockSpec((1, 1), lambda i: (0, 0)),
        out_shape=jax.ShapeDtypeStruct((1, 1), jnp.float32),
    )(c, c, lab, lab)
    return out[0, 0]


# speedup vs baseline: 18.4005x; 1.0011x over previous
import functools

import jax
import jax.numpy as jnp
from jax.experimental import pallas as pl

_TEMPERATURE = 0.07
_BASE_TEMPERATURE = 0.07


def _supcon_block_kernel(c_blk_ref, c_full_ref, lab_blk_ref, lab_full_ref,
                         out_ref, *, bm, n, inv_temp, scale):
    i = pl.program_id(0)
    c_blk = c_blk_ref[...]
    c_full = c_full_ref[...]
    dot = jax.lax.dot_general(
        c_blk, c_full, (((1,), (1,)), ((), ())),
        preferred_element_type=jnp.float32)
    t = (dot - 1.0) * inv_temp

    c_blk_f = c_blk.astype(jnp.float32)
    sq = jnp.sum(c_blk_f * c_blk_f, axis=1)
    t_self = (sq - 1.0) * inv_temp

    p = jnp.exp(t)
    denom = jnp.sum(p, axis=1) - jnp.exp(t_self)

    row_lab = lab_blk_ref[...].reshape(bm, 1)
    all_lab = lab_full_ref[...]
    maskeq = row_lab == all_lab

    sum_mask = jnp.sum(jnp.where(maskeq, 1.0, 0.0), axis=1) - 1.0
    sum_mt = jnp.sum(jnp.where(maskeq, t, 0.0), axis=1) - t_self

    per_row = sum_mt / sum_mask - jnp.log(denom)
    partial = jnp.reshape(-scale * jnp.sum(per_row), (1, 1))

    @pl.when(i == 0)
    def _init():
        out_ref[...] = partial

    @pl.when(i != 0)
    def _acc():
        out_ref[...] += partial


def kernel(features, labels):
    bsz, n_views, dim = features.shape
    n = bsz * n_views
    c = jnp.reshape(jnp.transpose(features, (1, 0, 2)), (n, dim))
    c = c.astype(jnp.bfloat16)
    lab = jnp.reshape(jnp.tile(labels, n_views), (1, n))

    bm = 256
    grid = n // bm
    scale = (_TEMPERATURE / _BASE_TEMPERATURE) / n

    body = functools.partial(
        _supcon_block_kernel, bm=bm, n=n,
        inv_temp=1.0 / _TEMPERATURE, scale=scale)

    out = pl.pallas_call(
        body,
        grid=(grid,),
        in_specs=[
            pl.BlockSpec((bm, dim), lambda i: (i, 0)),
            pl.BlockSpec((n, dim), lambda i: (0, 0)),
            pl.BlockSpec((1, bm), lambda i: (0, i)),
            pl.BlockSpec((1, n), lambda i: (0, 0)),
        ],
        out_specs=pl.BlockSpec((1, 1), lambda i: (0, 0)),
        out_shape=jax.ShapeDtypeStruct((1, 1), jnp.float32),
    )(c, c, lab, lab)
    return out[0, 0]


# speedup vs baseline: 25.2959x; 1.3747x over previous
import functools

import jax
import jax.numpy as jnp
from jax.experimental import pallas as pl
from jax.experimental.pallas import tpu as pltpu

_TEMPERATURE = 0.07
_BASE_TEMPERATURE = 0.07
_NUM_CLASSES = 128
_LOG2E = 1.4426950408889634
_LN2 = 0.6931471805599453


def _supcon_kernel(feats_ref, lab_blk_ref, lab_full_ref, out_ref,
                   cbf_ref, g_ref, cnt_ref, *, bm, n, dim, inv_temp, scale):
    i = pl.program_id(0)
    nc = _NUM_CLASSES

    @pl.when(i == 0)
    def _setup():
        cbf_ref[...] = feats_ref[...].astype(jnp.bfloat16)
        lab_all = lab_full_ref[...]
        class_ids = jax.lax.broadcasted_iota(jnp.int32, (nc, n), 0)
        eq = class_ids == lab_all
        e_bf = jnp.where(eq, 1.0, 0.0).astype(jnp.bfloat16)
        g = jax.lax.dot_general(
            e_bf, cbf_ref[...], (((1,), (0,)), ((), ())),
            preferred_element_type=jnp.float32)
        g_ref[...] = g.astype(jnp.bfloat16)
        cnt_ref[...] = jnp.sum(jnp.where(eq, 1.0, 0.0), axis=1,
                               keepdims=True)

    cb = cbf_ref[pl.ds(i * bm, bm), :]
    dot = jax.lax.dot_general(
        cb, cbf_ref[...], (((1,), (1,)), ((), ())),
        preferred_element_type=jnp.float32)

    a = inv_temp * _LOG2E
    p = jnp.exp2(dot * a - a)
    cb_f = cb.astype(jnp.float32)
    sq = jnp.sum(cb_f * cb_f, axis=1)
    denom = jnp.sum(p, axis=1) - jnp.exp2(sq * a - a)

    msums = jax.lax.dot_general(
        cb, g_ref[...], (((1,), (1,)), ((), ())),
        preferred_element_type=jnp.float32)
    row_lab = lab_blk_ref[...].reshape(bm, 1)
    col_ids = jax.lax.broadcasted_iota(jnp.int32, (bm, nc), 1)
    ohm = row_lab == col_ids
    msum = jnp.sum(jnp.where(ohm, msums, 0.0), axis=1)
    cnt = jax.lax.dot_general(
        jnp.where(ohm, 1.0, 0.0), cnt_ref[...],
        (((1,), (0,)), ((), ())),
        preferred_element_type=jnp.float32)[:, 0]

    sum_mt = (msum - sq - cnt + 1.0) * inv_temp
    per_row = sum_mt / (cnt - 1.0) - jnp.log(denom)
    partial = jnp.reshape(-scale * jnp.sum(per_row), (1, 1))

    @pl.when(i == 0)
    def _init():
        out_ref[...] = partial

    @pl.when(i != 0)
    def _acc():
        out_ref[...] += partial


def kernel(features, labels):
    bsz, n_views, dim = features.shape
    n = bsz * n_views
    feats2d = jnp.reshape(features, (n, dim))
    lab4 = jnp.reshape(
        jnp.broadcast_to(labels[:, None], (bsz, n_views)), (1, n))

    bm = 256
    grid = n // bm
    scale = (_TEMPERATURE / _BASE_TEMPERATURE) / n

    body = functools.partial(
        _supcon_kernel, bm=bm, n=n, dim=dim,
        inv_temp=1.0 / _TEMPERATURE, scale=scale)

    out = pl.pallas_call(
        body,
        grid=(grid,),
        in_specs=[
            pl.BlockSpec((n, dim), lambda i: (0, 0)),
            pl.BlockSpec((1, bm), lambda i: (0, i)),
            pl.BlockSpec((1, n), lambda i: (0, 0)),
        ],
        out_specs=pl.BlockSpec((1, 1), lambda i: (0, 0)),
        out_shape=jax.ShapeDtypeStruct((1, 1), jnp.float32),
        scratch_shapes=[
            pltpu.VMEM((n, dim), jnp.bfloat16),
            pltpu.VMEM((_NUM_CLASSES, dim), jnp.bfloat16),
            pltpu.VMEM((_NUM_CLASSES, 1), jnp.float32),
        ],
    )(feats2d, lab4, lab4)
    return out[0, 0]


# speedup vs baseline: 27.5718x; 1.0900x over previous
import functools

import jax
import jax.numpy as jnp
from jax.experimental import pallas as pl
from jax.experimental.pallas import tpu as pltpu

_TEMPERATURE = 0.07
_BASE_TEMPERATURE = 0.07
_NUM_CLASSES = 128
_LOG2E = 1.4426950408889634
_LN2 = 0.6931471805599453


def _supcon_kernel(feats_ref, lab_blk_ref, lab_full_ref, out_ref,
                   cbf_ref, g_ref, cnt_ref, *, bm, n, dim, inv_temp, scale):
    i = pl.program_id(0)
    nc = _NUM_CLASSES

    @pl.when(i == 0)
    def _setup():
        cbf_ref[...] = feats_ref[...].astype(jnp.bfloat16)
        lab_all = lab_full_ref[...]
        class_ids = jax.lax.broadcasted_iota(jnp.int32, (nc, n), 0)
        eq = class_ids == lab_all
        e_bf = jnp.where(eq, 1.0, 0.0).astype(jnp.bfloat16)
        g = jax.lax.dot_general(
            e_bf, cbf_ref[...], (((1,), (0,)), ((), ())),
            preferred_element_type=jnp.float32)
        g_ref[...] = g.astype(jnp.bfloat16)
        cnt_ref[...] = jnp.sum(jnp.where(eq, 1.0, 0.0), axis=1,
                               keepdims=True)

    cb = cbf_ref[pl.ds(i * bm, bm), :]
    dot = jax.lax.dot_general(
        cb, cbf_ref[...], (((1,), (1,)), ((), ())),
        preferred_element_type=jnp.float32)

    a = inv_temp * _LOG2E
    p = jnp.exp2(dot * a - a)
    cb_f = cb.astype(jnp.float32)
    sq = jnp.sum(cb_f * cb_f, axis=1)
    denom = jnp.sum(p, axis=1) - jnp.exp2(sq * a - a)

    msums = jax.lax.dot_general(
        cb, g_ref[...], (((1,), (1,)), ((), ())),
        preferred_element_type=jnp.float32)
    row_lab = lab_blk_ref[...].reshape(bm, 1)
    col_ids = jax.lax.broadcasted_iota(jnp.int32, (bm, nc), 1)
    ohm = row_lab == col_ids
    msum = jnp.sum(jnp.where(ohm, msums, 0.0), axis=1)
    cnt = jax.lax.dot_general(
        jnp.where(ohm, 1.0, 0.0), cnt_ref[...],
        (((1,), (0,)), ((), ())),
        preferred_element_type=jnp.float32)[:, 0]

    sum_mt = (msum - sq - cnt + 1.0) * inv_temp
    per_row = sum_mt / (cnt - 1.0) - jnp.log(denom)
    partial = jnp.reshape(-scale * jnp.sum(per_row), (1, 1))

    @pl.when(i == 0)
    def _init():
        out_ref[...] = partial

    @pl.when(i != 0)
    def _acc():
        out_ref[...] += partial


def kernel(features, labels):
    bsz, n_views, dim = features.shape
    n = bsz * n_views
    feats2d = jnp.reshape(features, (n, dim))
    lab4 = jnp.reshape(
        jnp.broadcast_to(labels[:, None], (bsz, n_views)), (1, n))

    bm = 512
    grid = n // bm
    scale = (_TEMPERATURE / _BASE_TEMPERATURE) / n

    body = functools.partial(
        _supcon_kernel, bm=bm, n=n, dim=dim,
        inv_temp=1.0 / _TEMPERATURE, scale=scale)

    out = pl.pallas_call(
        body,
        grid=(grid,),
        in_specs=[
            pl.BlockSpec((n, dim), lambda i: (0, 0)),
            pl.BlockSpec((1, bm), lambda i: (0, i)),
            pl.BlockSpec((1, n), lambda i: (0, 0)),
        ],
        out_specs=pl.BlockSpec((1, 1), lambda i: (0, 0)),
        out_shape=jax.ShapeDtypeStruct((1, 1), jnp.float32),
        scratch_shapes=[
            pltpu.VMEM((n, dim), jnp.bfloat16),
            pltpu.VMEM((_NUM_CLASSES, dim), jnp.bfloat16),
            pltpu.VMEM((_NUM_CLASSES, 1), jnp.float32),
        ],
    )(feats2d, lab4, lab4)
    return out[0, 0]


# speedup vs baseline: 29.0973x; 1.0553x over previous
import functools

import jax
import jax.numpy as jnp
from jax.experimental import pallas as pl
from jax.experimental.pallas import tpu as pltpu

_TEMPERATURE = 0.07
_BASE_TEMPERATURE = 0.07
_NUM_CLASSES = 128
_LOG2E = 1.4426950408889634
_LN2 = 0.6931471805599453


def _supcon_kernel(feats_ref, lab_blk_ref, lab_full_ref, out_ref,
                   cbf_ref, g_ref, cnt_ref, *, bm, n, dim, inv_temp, scale):
    i = pl.program_id(0)
    nc = _NUM_CLASSES

    @pl.when(i == 0)
    def _setup():
        cbf_ref[...] = feats_ref[...].astype(jnp.bfloat16)
        lab_all = lab_full_ref[...]
        class_ids = jax.lax.broadcasted_iota(jnp.int32, (nc, n), 0)
        eq = class_ids == lab_all
        e_bf = jnp.where(eq, 1.0, 0.0).astype(jnp.bfloat16)
        g = jax.lax.dot_general(
            e_bf, cbf_ref[...], (((1,), (0,)), ((), ())),
            preferred_element_type=jnp.float32)
        g_ref[...] = g.astype(jnp.bfloat16)
        cnt_ref[...] = jnp.sum(jnp.where(eq, 1.0, 0.0), axis=1,
                               keepdims=True)

    cb = cbf_ref[pl.ds(i * bm, bm), :]
    dot = jax.lax.dot_general(
        cb, cbf_ref[...], (((1,), (1,)), ((), ())),
        preferred_element_type=jnp.float32)

    a = inv_temp * _LOG2E
    p = jnp.exp2(dot * a - a)
    cb_f = cb.astype(jnp.float32)
    sq = jnp.sum(cb_f * cb_f, axis=1)
    denom = jnp.sum(p, axis=1) - jnp.exp2(sq * a - a)

    msums = jax.lax.dot_general(
        cb, g_ref[...], (((1,), (1,)), ((), ())),
        preferred_element_type=jnp.float32)
    row_lab = lab_blk_ref[...].reshape(bm, 1)
    col_ids = jax.lax.broadcasted_iota(jnp.int32, (bm, nc), 1)
    ohm = row_lab == col_ids
    msum = jnp.sum(jnp.where(ohm, msums, 0.0), axis=1)
    cnt = jax.lax.dot_general(
        jnp.where(ohm, 1.0, 0.0), cnt_ref[...],
        (((1,), (0,)), ((), ())),
        preferred_element_type=jnp.float32)[:, 0]

    sum_mt = (msum - sq - cnt + 1.0) * inv_temp
    per_row = sum_mt / (cnt - 1.0) - jnp.log(denom)
    partial = jnp.reshape(-scale * jnp.sum(per_row), (1, 1))

    @pl.when(i == 0)
    def _init():
        out_ref[...] = partial

    @pl.when(i != 0)
    def _acc():
        out_ref[...] += partial


def kernel(features, labels):
    bsz, n_views, dim = features.shape
    n = bsz * n_views
    feats2d = jnp.reshape(features, (n, dim))
    lab4 = jnp.reshape(
        jnp.broadcast_to(labels[:, None], (bsz, n_views)), (1, n))

    bm = 2048
    grid = n // bm
    scale = (_TEMPERATURE / _BASE_TEMPERATURE) / n

    body = functools.partial(
        _supcon_kernel, bm=bm, n=n, dim=dim,
        inv_temp=1.0 / _TEMPERATURE, scale=scale)

    out = pl.pallas_call(
        body,
        grid=(grid,),
        in_specs=[
            pl.BlockSpec((n, dim), lambda i: (0, 0)),
            pl.BlockSpec((1, bm), lambda i: (0, i)),
            pl.BlockSpec((1, n), lambda i: (0, 0)),
        ],
        out_specs=pl.BlockSpec((1, 1), lambda i: (0, 0)),
        out_shape=jax.ShapeDtypeStruct((1, 1), jnp.float32),
        scratch_shapes=[
            pltpu.VMEM((n, dim), jnp.bfloat16),
            pltpu.VMEM((_NUM_CLASSES, dim), jnp.bfloat16),
            pltpu.VMEM((_NUM_CLASSES, 1), jnp.float32),
        ],
    )(feats2d, lab4, lab4)
    return out[0, 0]
